# Initial kernel scaffold; baseline (speedup 1.0000x reference)
import jax
import jax.numpy as jnp
from jax.experimental import pallas as pl


def kernel(x, edge_attr, W_l1, b_l1, W_r1, b_r1, W_e1, att1, bias1, gn_w, gn_b, gn_ms, W_l3, b_l3, W_r3, fc1_w, fc1_b, fc2_w, fc2_b, fc3_w, fc3_b, edge_index, batch):
    raise NotImplementedError("write your pallas kernel here")


# speedup vs baseline: 5.1571x; 5.1571x over previous
import functools
import numpy as np
import jax
import jax.numpy as jnp
from jax.experimental import pallas as pl

N = 50000
E = 800000
H = 16
C = 4
HC = 64
B = 64

EB = 4000
GE = E // EB
NB = 2000
GN = N // NB

_MASK_JH = np.equal(np.arange(HC)[:, None] // C, np.arange(H)[None, :]).astype(np.float32)


def _k1(xs_ref, xd_ref, ea_ref, wl_ref, bl_ref, wr_ref, br_ref, we_ref, A_ref, araw_ref):
    xs = xs_ref[0, 0, :]
    xd = xd_ref[0, 0, :]
    ea = ea_ref[0, 0, :]
    wl = wl_ref[0, :]
    bl = bl_ref[0, :]
    wr = wr_ref[0, :]
    br = br_ref[0, :]
    we = we_ref[0, :]
    hi = (xs[:, None] * wl[None, :] + bl[None, :]
          + xd[:, None] * wr[None, :] + br[None, :]
          + ea[:, None] * we[None, :])
    hi = jnp.where(hi >= 0, hi, 0.2 * hi)
    araw_ref[...] = jnp.dot(hi, A_ref[...], preferred_element_type=jnp.float32)


def _k2(araw_ref, amaxd_ref, xs_ref, aexp_ref, aexs_ref):
    a = jnp.exp(araw_ref[...] - amaxd_ref[...])
    aexp_ref[...] = a
    aexs_ref[...] = a * xs_ref[0, 0, :][:, None]


def _k4(s0_ref, s1_ref, wl_ref, bl_ref, bias_ref, R_ref, batch_ref,
        h1_ref, sumh_ref, sumh2_ref, cnt_ref):
    s0r = jnp.dot(s0_ref[...], R_ref[...], preferred_element_type=jnp.float32)
    s1r = jnp.dot(s1_ref[...], R_ref[...], preferred_element_type=jnp.float32)
    h1 = s1r * wl_ref[0, :][None, :] + s0r * bl_ref[0, :][None, :] + bias_ref[0, :][None, :]
    h1 = jnp.maximum(h1, 0.0)
    h1_ref[...] = h1
    b = batch_ref[0, 0, :]
    oh = (b[:, None] == jax.lax.broadcasted_iota(jnp.int32, (NB, B), 1)).astype(jnp.float32)

    @pl.when(pl.program_id(0) == 0)
    def _init():
        sumh_ref[...] = jnp.zeros_like(sumh_ref)
        sumh2_ref[...] = jnp.zeros_like(sumh2_ref)
        cnt_ref[...] = jnp.zeros_like(cnt_ref)

    dn = (((0,), (0,)), ((), ()))
    sumh_ref[...] += jax.lax.dot_general(oh, h1, dn, preferred_element_type=jnp.float32)
    sumh2_ref[...] += jax.lax.dot_general(oh, h1 * h1, dn, preferred_element_type=jnp.float32)
    cnt_ref[...] += jax.lax.dot_general(oh, jnp.ones((NB, HC), jnp.float32), dn,
                                        preferred_element_type=jnp.float32)


def _k5(h1_ref, batch_ref, sumh_ref, sumh2_ref, cnt_ref, ms_ref, gw_ref, gb_ref, hn_ref):
    cnt = jnp.maximum(cnt_ref[...], 1.0)
    mean = sumh_ref[...] / cnt
    m = ms_ref[0, :][None, :]
    mm = m * mean
    var = (sumh2_ref[...] - 2.0 * mm * sumh_ref[...] + cnt_ref[...] * mm * mm) / cnt
    b = batch_ref[0, 0, :]
    oh = (b[:, None] == jax.lax.broadcasted_iota(jnp.int32, (NB, B), 1)).astype(jnp.float32)
    mean_g = jnp.dot(oh, mm, preferred_element_type=jnp.float32)
    var_g = jnp.dot(oh, var, preferred_element_type=jnp.float32)
    sub = h1_ref[...] - mean_g
    hn_ref[...] = sub * jax.lax.rsqrt(var_g + 1e-5) * gw_ref[0, :][None, :] + gb_ref[0, :][None, :]


def _k6(aggs_ref, deg_ref, hn_ref, wl3_ref, bl3_ref, wr3_ref, batch_ref,
        sum2_ref, max2_ref):
    deg = jnp.maximum(deg_ref[0, 0, :], 1.0)
    aggr = aggs_ref[...] / deg[:, None]
    h2 = (jnp.dot(aggr, wl3_ref[...], preferred_element_type=jnp.float32)
          + bl3_ref[0, :][None, :]
          + jnp.dot(hn_ref[...], wr3_ref[...], preferred_element_type=jnp.float32))
    h2 = jnp.maximum(h2, 0.0)
    b2 = batch_ref[0, 0, :][:, None]
    oh = (b2 == jax.lax.broadcasted_iota(jnp.int32, (NB, B), 1)).astype(jnp.float32)

    @pl.when(pl.program_id(0) == 0)
    def _init():
        sum2_ref[...] = jnp.zeros_like(sum2_ref)
        max2_ref[...] = jnp.full_like(max2_ref, -1e30)

    dn = (((0,), (0,)), ((), ()))
    sum2_ref[...] += jax.lax.dot_general(oh, h2, dn, preferred_element_type=jnp.float32)
    rows = []
    for seg in range(B):
        mask = b2 == seg
        rows.append(jnp.max(jnp.where(mask, h2, -1e30), axis=0, keepdims=True))
    stacked = jnp.concatenate(rows, axis=0)
    max2_ref[...] = jnp.maximum(max2_ref[...], stacked)


def _k7(sum2_ref, max2_ref, cnt_ref, w1_ref, b1_ref, w2_ref, b2_ref, w3_ref, b3_ref, out_ref):
    cnt_raw = cnt_ref[...]
    x1 = jnp.where(cnt_raw > 0, max2_ref[...], 0.0)
    x2 = sum2_ref[...] / jnp.maximum(cnt_raw, 1.0)
    z = jnp.concatenate([x1, x2], axis=1)
    z = jnp.maximum(jnp.dot(z, w1_ref[...], preferred_element_type=jnp.float32)
                    + b1_ref[0, :][None, :], 0.0)
    z = jnp.maximum(jnp.dot(z, w2_ref[...], preferred_element_type=jnp.float32)
                    + b2_ref[0, :][None, :], 0.0)
    out_ref[...] = (jnp.dot(z, w3_ref[...], preferred_element_type=jnp.float32)
                    + b3_ref[0, :][None, :])


def _eb1(i):
    return (i, 0, 0)


def _eb2(i):
    return (i, 0)


def _zero2(i):
    return (0, 0)


def _zero3(i):
    return (0, 0, 0)


def _nb2(i):
    return (i, 0)


@jax.jit
def kernel(x, edge_attr, W_l1, b_l1, W_r1, b_r1, W_e1, att1, bias1, gn_w, gn_b, gn_ms,
           W_l3, b_l3, W_r3, fc1_w, fc1_b, fc2_w, fc2_b, fc3_w, fc3_b, edge_index, batch):
    src, dst = edge_index[0], edge_index[1]
    xs = x[src]
    xd = x[dst]

    xs3 = xs.reshape(GE, 1, EB)
    xd3 = xd.reshape(GE, 1, EB)
    ea3 = edge_attr.reshape(GE, 1, EB)
    wl = W_l1.reshape(1, HC)
    bl = b_l1.reshape(1, HC)
    wr = W_r1.reshape(1, HC)
    br = b_r1.reshape(1, HC)
    we = W_e1.reshape(1, HC)
    mask_jh = jnp.asarray(_MASK_JH)
    A = att1.reshape(HC, 1) * mask_jh
    R = mask_jh.T

    espec1 = pl.BlockSpec((1, 1, EB), _eb1)
    espec16 = pl.BlockSpec((EB, H), _eb2)
    w1spec = pl.BlockSpec((1, HC), _zero2)

    araw = pl.pallas_call(
        _k1,
        grid=(GE,),
        in_specs=[espec1, espec1, espec1, w1spec, w1spec, w1spec, w1spec, w1spec,
                  pl.BlockSpec((HC, H), _zero2)],
        out_specs=espec16,
        out_shape=jax.ShapeDtypeStruct((E, H), jnp.float32),
    )(xs3, xd3, ea3, wl, bl, wr, br, we, A)

    amax = jax.ops.segment_max(araw, dst, N)
    amax = jnp.where(jnp.isfinite(amax), amax, 0.0)
    amaxd = amax[dst]

    aexp, aexs = pl.pallas_call(
        _k2,
        grid=(GE,),
        in_specs=[espec16, espec16, espec1],
        out_specs=[espec16, espec16],
        out_shape=[jax.ShapeDtypeStruct((E, H), jnp.float32),
                   jax.ShapeDtypeStruct((E, H), jnp.float32)],
    )(araw, amaxd, xs3)

    asum = jax.ops.segment_sum(aexp, dst, N)
    t1 = jax.ops.segment_sum(aexs, dst, N)
    denom = asum + 1e-16
    S0 = asum / denom
    S1 = t1 / denom

    batch3 = batch.reshape(GN, 1, NB)
    nspec16 = pl.BlockSpec((NB, H), _nb2)
    nspec64 = pl.BlockSpec((NB, HC), _nb2)
    bspec = pl.BlockSpec((1, 1, NB), _eb1)
    statspec = pl.BlockSpec((B, HC), _zero2)

    h1, sumh, sumh2, cntm = pl.pallas_call(
        _k4,
        grid=(GN,),
        in_specs=[nspec16, nspec16, w1spec, w1spec, w1spec,
                  pl.BlockSpec((H, HC), _zero2), bspec],
        out_specs=[nspec64, statspec, statspec, statspec],
        out_shape=[jax.ShapeDtypeStruct((N, HC), jnp.float32),
                   jax.ShapeDtypeStruct((B, HC), jnp.float32),
                   jax.ShapeDtypeStruct((B, HC), jnp.float32),
                   jax.ShapeDtypeStruct((B, HC), jnp.float32)],
    )(S0, S1, wl, bl, bias1.reshape(1, HC), R, batch3)

    hn = pl.pallas_call(
        _k5,
        grid=(GN,),
        in_specs=[nspec64, bspec, statspec, statspec, statspec,
                  w1spec, w1spec, w1spec],
        out_specs=nspec64,
        out_shape=jax.ShapeDtypeStruct((N, HC), jnp.float32),
    )(h1, batch3, sumh, sumh2, cntm, gn_ms.reshape(1, HC), gn_w.reshape(1, HC),
      gn_b.reshape(1, HC))

    aggs = jax.ops.segment_sum(hn[src], dst, N)
    deg = jax.ops.segment_sum(jnp.ones((E,), jnp.float32), dst, N)
    deg3 = deg.reshape(GN, 1, NB)

    sum2, max2 = pl.pallas_call(
        _k6,
        grid=(GN,),
        in_specs=[nspec64, bspec, nspec64,
                  pl.BlockSpec((HC, HC), _zero2), w1spec, pl.BlockSpec((HC, HC), _zero2),
                  bspec],
        out_specs=[statspec, statspec],
        out_shape=[jax.ShapeDtypeStruct((B, HC), jnp.float32),
                   jax.ShapeDtypeStruct((B, HC), jnp.float32)],
    )(aggs, deg3, hn, W_l3, b_l3.reshape(1, HC), W_r3, batch3)

    fc3p = jnp.zeros((1024, 128), jnp.float32).at[:, :3].set(fc3_w)
    fc3bp = jnp.zeros((1, 128), jnp.float32).at[0, :3].set(fc3_b)

    outp = pl.pallas_call(
        _k7,
        grid=(1,),
        in_specs=[statspec, statspec, statspec,
                  pl.BlockSpec((2 * HC, 2048), _zero2), pl.BlockSpec((1, 2048), _zero2),
                  pl.BlockSpec((2048, 1024), _zero2), pl.BlockSpec((1, 1024), _zero2),
                  pl.BlockSpec((1024, 128), _zero2), pl.BlockSpec((1, 128), _zero2)],
        out_specs=pl.BlockSpec((B, 128), _zero2),
        out_shape=jax.ShapeDtypeStruct((B, 128), jnp.float32),
    )(sum2, max2, cntm, fc1_w, fc1_b.reshape(1, 2048), fc2_w, fc2_b.reshape(1, 1024),
      fc3p, fc3bp)

    return outp[:, :3]
